# baseline (speedup 1.0000x reference)
import functools

import jax
import jax.numpy as jnp
import numpy as np
from jax import lax
from jax.experimental import pallas as pl
from jax.experimental.pallas import tpu as pltpu
from jax.experimental.pallas import tpu_sc as plsc

_FIELD_DIMS = np.array([100000] * 26, dtype=np.int64)
_OFFSETS = np.concatenate(([0], np.cumsum(_FIELD_DIMS)[:-1])).astype(np.int32)

_NC = 2
_NS = 16
_NW = _NC * _NS
_BATCH = 4096
_NF = 26
_D = 32
_TOTAL = _BATCH * _NF
_PER_W = _TOTAL // _NW
_CHUNK = 128
_NCHUNK = _PER_W // _CHUNK

_OFFS_TILED = _OFFSETS[(np.arange(_PER_W) % _NF)].reshape(_NCHUNK, _CHUNK)


def _body(x_hbm, offs_hbm, w_hbm, out_hbm, idx_v, offs_v, rows_v, sem):
    c = lax.axis_index("c")
    s = lax.axis_index("s")
    wid = s * _NC + c

    pltpu.sync_copy(x_hbm.at[wid], idx_v)
    pltpu.sync_copy(offs_hbm, offs_v)

    def add_body(i, carry):
        j = i // (_CHUNK // 16)
        k = (i % (_CHUNK // 16)) * 16
        idx_v[j, pl.ds(k, 16)] = idx_v[j, pl.ds(k, 16)] + offs_v[j, pl.ds(k, 16)]
        return carry

    lax.fori_loop(0, _NCHUNK * (_CHUNK // 16), add_body, 0)

    copies = [
        pltpu.async_copy(
            w_hbm.at[idx_v.at[j]],
            rows_v.at[pl.ds(j * _CHUNK, _CHUNK)],
            sem,
        )
        for j in range(_NCHUNK)
    ]
    for cp in copies:
        cp.wait()

    pltpu.sync_copy(rows_v, out_hbm.at[wid])


@jax.jit
def kernel(x, W):
    mesh = plsc.VectorSubcoreMesh(
        core_axis_name="c", subcore_axis_name="s", num_cores=_NC, num_subcores=_NS
    )
    x3 = x.reshape(_NW, _NCHUNK, _CHUNK)
    offs = jnp.asarray(_OFFS_TILED)
    w16 = W.astype(jnp.bfloat16)
    out = pl.kernel(
        _body,
        out_type=jax.ShapeDtypeStruct((_NW, _PER_W, _D), jnp.bfloat16),
        mesh=mesh,
        scratch_types=[
            pltpu.VMEM((_NCHUNK, _CHUNK), jnp.int32),
            pltpu.VMEM((_NCHUNK, _CHUNK), jnp.int32),
            pltpu.VMEM((_PER_W, _D), jnp.bfloat16),
            pltpu.SemaphoreType.DMA,
        ],
        compiler_params=pltpu.CompilerParams(use_tc_tiling_on_sc=False),
    )(x3, offs, w16)
    return out.reshape(_BATCH, _NF, _D).astype(jnp.float32)


# speedup vs baseline: 1.2173x; 1.2173x over previous
import functools

import jax
import jax.numpy as jnp
import numpy as np
from jax import lax
from jax.experimental import pallas as pl
from jax.experimental.pallas import tpu as pltpu
from jax.experimental.pallas import tpu_sc as plsc

_FIELD_DIMS = np.array([100000] * 26, dtype=np.int64)
_OFFSETS = np.concatenate(([0], np.cumsum(_FIELD_DIMS)[:-1])).astype(np.int32)

_NC = 2
_NS = 16
_NW = _NC * _NS
_BATCH = 4096
_NF = 26
_D = 32
_TOTAL = _BATCH * _NF
_PER_W = _TOTAL // _NW
_CHUNK = 128
_NCHUNK = _PER_W // _CHUNK

_OFFS_TILED = _OFFSETS[(np.arange(_PER_W) % _NF)].reshape(_NCHUNK, _CHUNK)


def _body(x_hbm, offs_hbm, w_hbm, out_hbm, idx_v, offs_v, rows_v, sem):
    c = lax.axis_index("c")
    s = lax.axis_index("s")
    wid = s * _NC + c

    pltpu.sync_copy(x_hbm.at[wid], idx_v)
    pltpu.sync_copy(offs_hbm, offs_v)

    def add_body(i, carry):
        j = i // (_CHUNK // 16)
        k = (i % (_CHUNK // 16)) * 16
        idx_v[j, pl.ds(k, 16)] = idx_v[j, pl.ds(k, 16)] + offs_v[j, pl.ds(k, 16)]
        return carry

    lax.fori_loop(0, _NCHUNK * (_CHUNK // 16), add_body, 0)

    copies = [
        pltpu.async_copy(
            w_hbm.at[idx_v.at[j]],
            rows_v.at[pl.ds(j * _CHUNK, _CHUNK)],
            sem,
        )
        for j in range(_NCHUNK)
    ]
    for cp in copies:
        cp.wait()

    pltpu.sync_copy(rows_v, out_hbm.at[wid])


@jax.jit
def kernel(x, W):
    mesh = plsc.VectorSubcoreMesh(
        core_axis_name="c", subcore_axis_name="s", num_cores=_NC, num_subcores=_NS
    )
    x3 = x.reshape(_NW, _NCHUNK, _CHUNK)
    offs = jnp.asarray(_OFFS_TILED)
    out = pl.kernel(
        _body,
        out_type=jax.ShapeDtypeStruct((_NW, _PER_W, _D), jnp.float32),
        mesh=mesh,
        scratch_types=[
            pltpu.VMEM((_NCHUNK, _CHUNK), jnp.int32),
            pltpu.VMEM((_NCHUNK, _CHUNK), jnp.int32),
            pltpu.VMEM((_PER_W, _D), jnp.float32),
            pltpu.SemaphoreType.DMA,
        ],
        compiler_params=pltpu.CompilerParams(use_tc_tiling_on_sc=False),
    )(x3, offs, W)
    return out.reshape(_BATCH, _NF, _D)


# speedup vs baseline: 3.4822x; 2.8607x over previous
import jax
import jax.numpy as jnp
import numpy as np
from jax import lax
from jax.experimental import pallas as pl
from jax.experimental.pallas import tpu as pltpu
from jax.experimental.pallas import tpu_sc as plsc

_NC = 2
_NS = 16
_NW = _NC * _NS
_BATCH = 4096
_NF = 26
_NV = 2_600_000
_BPW = _BATCH // _NW
_WIDTH = 1152
_BUFW = 1280
_SMAX = _NV - _WIDTH - 64
_TAIL0 = _NV - 64


def _sel(ref_1d, pos, i16):
    ch = ref_1d[pl.ds((pos >> 4) << 4, 16)]
    return jnp.sum(jnp.where(i16 == (pos & 15), ch, 0))


def _body1(w3, xst, pst, wtail, packed, ridp, xv, pv, wt_v, rid_v, bvm,
           dlist, starts, bufs, outf, sem):
    c = lax.axis_index("c")
    s = lax.axis_index("s")
    wid = s * _NC + c
    base = wid * _BPW

    pltpu.sync_copy(xst.at[:, pl.ds(base, _BPW)], xv)
    pltpu.sync_copy(pst.at[:, pl.ds(base, _BPW)], pv)
    pltpu.sync_copy(wtail, wt_v)

    i16 = lax.iota(jnp.int32, 16)
    p_lo = lax.shift_right_logical(i16, 3)
    p_hi = p_lo + 2
    r_id = lax.bitwise_and(i16, 7)

    def floop(f, carry):
        off = f * 100000

        def ridloop(g, c2):
            rid_v[pl.ds(g * 16, 16)] = pv[f, pl.ds(g * 16, 16)] * _NF + f
            return c2

        lax.fori_loop(0, _BPW // 16, ridloop, 0)
        pltpu.sync_copy(rid_v, ridp.at[f, wid, :])

        v0 = xv[f, pl.ds(0, 16)][0] + off
        lo = lax.shift_left(lax.shift_right_logical(v0, 10), 10)

        bvm[pl.ds(0, 16)] = jnp.full((16,), -1, jnp.int32)

        def cidloop(g, c2):
            vv = xv[f, pl.ds(g * 16, 16)] + off
            bvm[pl.ds(1 + g * 16, 16)] = lax.shift_right_logical(vv - lo, 10)
            return c2

        lax.fori_loop(0, _BPW // 16, cidloop, 0)

        ptr = jnp.int32(0)
        for g in range(_BPW // 16):
            prev = bvm[pl.ds(g * 16, 16)]
            cur = bvm[pl.ds(g * 16 + 1, 16)]
            m = cur != prev
            plsc.store_compressed(dlist.at[pl.ds(ptr, 16)], cur, mask=m)
            plsc.store_compressed(
                starts.at[pl.ds(ptr, 16)], i16 + g * 16, mask=m
            )
            ptr = ptr + plsc.all_reduce_population_count(m)[0]
        ndist = ptr
        starts[pl.ds(ndist, 16)] = jnp.full((16,), _BPW, jnp.int32)

        s_0 = pl.multiple_of(
            jnp.minimum(lo + _sel(dlist, jnp.int32(0), i16) * 1024, _SMAX), 128
        )
        pltpu.async_copy(
            w3.at[:, :, pl.ds(s_0, _WIDTH)],
            bufs.at[0, :, :, pl.ds(0, _WIDTH)],
            sem,
        )

        def dloop(d, c2):
            pltpu.make_async_copy(
                w3.at[:, :, pl.ds(0, _WIDTH)],
                bufs.at[d % 2, :, :, pl.ds(0, _WIDTH)],
                sem,
            ).wait()

            @pl.when(d + 1 < ndist)
            def _():
                sn = pl.multiple_of(
                    jnp.minimum(lo + _sel(dlist, d + 1, i16) * 1024, _SMAX),
                    128,
                )
                pltpu.async_copy(
                    w3.at[:, :, pl.ds(sn, _WIDTH)],
                    bufs.at[(d + 1) % 2, :, :, pl.ds(0, _WIDTH)],
                    sem,
                )

            sd = pl.multiple_of(
                jnp.minimum(lo + _sel(dlist, d, i16) * 1024, _SMAX), 128
            )
            r0 = _sel(starts, d, i16)
            r1 = _sel(starts, d + 1, i16)

            def rloop(r, c3):
                ch = xv[f, pl.ds((r >> 4) << 4, 16)]
                v = jnp.sum(jnp.where(i16 == (r & 15), ch, 0)) + off
                colv = i16 * 0 + (v - sd)
                g0 = plsc.load_gather(bufs.at[d % 2], [p_lo, r_id, colv])
                g1 = plsc.load_gather(bufs.at[d % 2], [p_hi, r_id, colv])
                tl = jnp.minimum(jnp.maximum(v - _TAIL0, 0), 63)
                t0 = wt_v[tl, pl.ds(0, 16)]
                t1 = wt_v[tl, pl.ds(16, 16)]
                mt = i16 * 0 + jnp.where(v >= _TAIL0, 1, 0)
                g0 = jnp.where(mt == 1, t0, g0)
                g1 = jnp.where(mt == 1, t1, g1)
                row32 = r >> 2
                cb = (r & 3) * 32
                outf[row32, pl.ds(cb, 16)] = g0
                outf[row32, pl.ds(cb + 16, 16)] = g1
                return c3

            lax.fori_loop(r0, r1, rloop, 0)
            return c2

        lax.fori_loop(0, ndist, dloop, 0)
        pltpu.sync_copy(outf, packed.at[f, pl.ds(wid * 32, 32), :])
        return carry

    lax.fori_loop(0, _NF, floop, 0)


def _body2(packed, ridp, res, slab_v, rows_v, rid_v, sem):
    c = lax.axis_index("c")
    s = lax.axis_index("s")
    wid = s * _NC + c

    def floop(f, carry):
        pltpu.sync_copy(packed.at[f, pl.ds(wid * 32, 32), :], slab_v)
        pltpu.sync_copy(ridp.at[f, wid, :], rid_v)

        def mv(t, c2):
            r = t >> 1
            half = lax.bitwise_and(t, 1) * 16
            rows_v[r, pl.ds(half, 16)] = slab_v[
                r >> 2, pl.ds((lax.bitwise_and(r, 3)) * 32 + half, 16)
            ]
            return c2

        lax.fori_loop(0, 256, mv, 0)
        pltpu.async_copy(rows_v, res.at[rid_v], sem).wait()
        return carry

    lax.fori_loop(0, _NF, floop, 0)


@jax.jit
def kernel(x, W):
    mesh = plsc.VectorSubcoreMesh(
        core_axis_name="c", subcore_axis_name="s", num_cores=_NC, num_subcores=_NS
    )
    iota_b = lax.broadcasted_iota(jnp.int32, x.shape, 0)
    xs, perm = lax.sort((x, iota_b), dimension=0, num_keys=1)
    w3 = W.T.reshape(4, 8, _NV)
    wtail = W[_TAIL0:, :]

    packed, ridp = pl.kernel(
        _body1,
        out_type=(
            jax.ShapeDtypeStruct((_NF, 1024, 128), jnp.float32),
            jax.ShapeDtypeStruct((_NF, _NW, _BPW), jnp.int32),
        ),
        mesh=mesh,
        scratch_types=[
            pltpu.VMEM((_NF, _BPW), jnp.int32),
            pltpu.VMEM((_NF, _BPW), jnp.int32),
            pltpu.VMEM((64, 32), jnp.float32),
            pltpu.VMEM((_BPW,), jnp.int32),
            pltpu.VMEM((144,), jnp.int32),
            pltpu.VMEM((160,), jnp.int32),
            pltpu.VMEM((160,), jnp.int32),
            pltpu.VMEM((2, 4, 8, _BUFW), jnp.float32),
            pltpu.VMEM((32, 128), jnp.float32),
            pltpu.SemaphoreType.DMA,
        ],
        compiler_params=pltpu.CompilerParams(
            use_tc_tiling_on_sc=True, needs_layout_passes=False
        ),
    )(w3, xs.T, perm.T, wtail)

    res = pl.kernel(
        _body2,
        out_type=jax.ShapeDtypeStruct((_BATCH * _NF, 32), jnp.float32),
        mesh=mesh,
        scratch_types=[
            pltpu.VMEM((32, 128), jnp.float32),
            pltpu.VMEM((_BPW, 32), jnp.float32),
            pltpu.VMEM((_BPW,), jnp.int32),
            pltpu.SemaphoreType.DMA,
        ],
        compiler_params=pltpu.CompilerParams(
            use_tc_tiling_on_sc=False, needs_layout_passes=False
        ),
    )(packed, ridp)

    return res.reshape(_BATCH, _NF, 32)


# speedup vs baseline: 3.5037x; 1.0062x over previous
import jax
import jax.numpy as jnp
import numpy as np
from jax import lax
from jax.experimental import pallas as pl
from jax.experimental.pallas import tpu as pltpu
from jax.experimental.pallas import tpu_sc as plsc

_NC = 2
_NS = 16
_NW = _NC * _NS
_BATCH = 4096
_NF = 26
_NV = 2_600_000
_BPW = _BATCH // _NW
_WIDTH = 1152
_BUFW = 1280
_SMAX = _NV - _WIDTH - 64
_TAIL0 = _NV - 64


def _sel(ref_1d, pos, i16):
    ch = ref_1d[pl.ds((pos >> 4) << 4, 16)]
    return jnp.sum(jnp.where(i16 == (pos & 15), ch, 0))


def _body1(w3, xst, pst, wtail, packed, ridp, xv, pv, wt_v, rid_v, bvm,
           dlist, starts, bufs, outf, sem):
    c = lax.axis_index("c")
    s = lax.axis_index("s")
    wid = s * _NC + c
    base = wid * _BPW

    pltpu.sync_copy(xst.at[:, pl.ds(base, _BPW)], xv)
    pltpu.sync_copy(pst.at[:, pl.ds(base, _BPW)], pv)
    pltpu.sync_copy(wtail, wt_v)

    i16 = lax.iota(jnp.int32, 16)
    p_lo = lax.shift_right_logical(i16, 3)
    p_hi = p_lo + 2
    r_id = lax.bitwise_and(i16, 7)

    def floop(f, carry):
        off = f * 100000

        def ridloop(g, c2):
            rid_v[pl.ds(g * 16, 16)] = pv[f, pl.ds(g * 16, 16)] * _NF + f
            return c2

        lax.fori_loop(0, _BPW // 16, ridloop, 0, unroll=4)
        pltpu.sync_copy(rid_v, ridp.at[f, wid, :])

        v0 = xv[f, pl.ds(0, 16)][0] + off
        lo = lax.shift_left(lax.shift_right_logical(v0, 10), 10)

        bvm[pl.ds(0, 16)] = jnp.full((16,), -1, jnp.int32)

        def cidloop(g, c2):
            vv = xv[f, pl.ds(g * 16, 16)] + off
            bvm[pl.ds(1 + g * 16, 16)] = lax.shift_right_logical(vv - lo, 10)
            return c2

        lax.fori_loop(0, _BPW // 16, cidloop, 0, unroll=4)

        ptr = jnp.int32(0)
        for g in range(_BPW // 16):
            prev = bvm[pl.ds(g * 16, 16)]
            cur = bvm[pl.ds(g * 16 + 1, 16)]
            m = cur != prev
            plsc.store_compressed(dlist.at[pl.ds(ptr, 16)], cur, mask=m)
            plsc.store_compressed(
                starts.at[pl.ds(ptr, 16)], i16 + g * 16, mask=m
            )
            ptr = ptr + plsc.all_reduce_population_count(m)[0]
        ndist = ptr
        starts[pl.ds(ndist, 16)] = jnp.full((16,), _BPW, jnp.int32)

        s_0 = pl.multiple_of(
            jnp.minimum(lo + _sel(dlist, jnp.int32(0), i16) * 1024, _SMAX), 128
        )
        pltpu.async_copy(
            w3.at[:, :, pl.ds(s_0, _WIDTH)],
            bufs.at[0, :, :, pl.ds(0, _WIDTH)],
            sem,
        )

        def dloop(d, c2):
            pltpu.make_async_copy(
                w3.at[:, :, pl.ds(0, _WIDTH)],
                bufs.at[d % 2, :, :, pl.ds(0, _WIDTH)],
                sem,
            ).wait()

            @pl.when(d + 1 < ndist)
            def _():
                sn = pl.multiple_of(
                    jnp.minimum(lo + _sel(dlist, d + 1, i16) * 1024, _SMAX),
                    128,
                )
                pltpu.async_copy(
                    w3.at[:, :, pl.ds(sn, _WIDTH)],
                    bufs.at[(d + 1) % 2, :, :, pl.ds(0, _WIDTH)],
                    sem,
                )

            sd = pl.multiple_of(
                jnp.minimum(lo + _sel(dlist, d, i16) * 1024, _SMAX), 128
            )
            r0 = _sel(starts, d, i16)
            r1 = _sel(starts, d + 1, i16)

            def rloop(r, c3):
                ch = xv[f, pl.ds((r >> 4) << 4, 16)]
                v = jnp.sum(jnp.where(i16 == (r & 15), ch, 0)) + off
                colv = i16 * 0 + (v - sd)
                g0 = plsc.load_gather(bufs.at[d % 2], [p_lo, r_id, colv])
                g1 = plsc.load_gather(bufs.at[d % 2], [p_hi, r_id, colv])
                tl = jnp.minimum(jnp.maximum(v - _TAIL0, 0), 63)
                t0 = wt_v[tl, pl.ds(0, 16)]
                t1 = wt_v[tl, pl.ds(16, 16)]
                mt = i16 * 0 + jnp.where(v >= _TAIL0, 1, 0)
                g0 = jnp.where(mt == 1, t0, g0)
                g1 = jnp.where(mt == 1, t1, g1)
                row32 = r >> 2
                cb = (r & 3) * 32
                outf[row32, pl.ds(cb, 16)] = g0
                outf[row32, pl.ds(cb + 16, 16)] = g1
                return c3

            lax.fori_loop(r0, r1, rloop, 0)
            return c2

        lax.fori_loop(0, ndist, dloop, 0)
        pltpu.sync_copy(outf, packed.at[f, pl.ds(wid * 32, 32), :])
        return carry

    lax.fori_loop(0, _NF, floop, 0)


def _body2(packed, ridp, res, slab_v, rows_v, rid_v, sem):
    c = lax.axis_index("c")
    s = lax.axis_index("s")
    wid = s * _NC + c

    def floop(f, carry):
        pltpu.sync_copy(packed.at[f, pl.ds(wid * 32, 32), :], slab_v)
        pltpu.sync_copy(ridp.at[f, wid, :], rid_v)

        def mv(t, c2):
            r = t >> 1
            half = lax.bitwise_and(t, 1) * 16
            rows_v[r, pl.ds(half, 16)] = slab_v[
                r >> 2, pl.ds((lax.bitwise_and(r, 3)) * 32 + half, 16)
            ]
            return c2

        lax.fori_loop(0, 256, mv, 0, unroll=8)
        pltpu.async_copy(rows_v, res.at[rid_v], sem).wait()
        return carry

    lax.fori_loop(0, _NF, floop, 0)


@jax.jit
def kernel(x, W):
    mesh = plsc.VectorSubcoreMesh(
        core_axis_name="c", subcore_axis_name="s", num_cores=_NC, num_subcores=_NS
    )
    iota_b = lax.broadcasted_iota(jnp.int32, x.shape, 0)
    xs, perm = lax.sort((x, iota_b), dimension=0, num_keys=1)
    w3 = W.T.reshape(4, 8, _NV)
    wtail = W[_TAIL0:, :]

    packed, ridp = pl.kernel(
        _body1,
        out_type=(
            jax.ShapeDtypeStruct((_NF, 1024, 128), jnp.float32),
            jax.ShapeDtypeStruct((_NF, _NW, _BPW), jnp.int32),
        ),
        mesh=mesh,
        scratch_types=[
            pltpu.VMEM((_NF, _BPW), jnp.int32),
            pltpu.VMEM((_NF, _BPW), jnp.int32),
            pltpu.VMEM((64, 32), jnp.float32),
            pltpu.VMEM((_BPW,), jnp.int32),
            pltpu.VMEM((144,), jnp.int32),
            pltpu.VMEM((160,), jnp.int32),
            pltpu.VMEM((160,), jnp.int32),
            pltpu.VMEM((2, 4, 8, _BUFW), jnp.float32),
            pltpu.VMEM((32, 128), jnp.float32),
            pltpu.SemaphoreType.DMA,
        ],
        compiler_params=pltpu.CompilerParams(
            use_tc_tiling_on_sc=True, needs_layout_passes=False
        ),
    )(w3, xs.T, perm.T, wtail)

    res = pl.kernel(
        _body2,
        out_type=jax.ShapeDtypeStruct((_BATCH * _NF, 32), jnp.float32),
        mesh=mesh,
        scratch_types=[
            pltpu.VMEM((32, 128), jnp.float32),
            pltpu.VMEM((_BPW, 32), jnp.float32),
            pltpu.VMEM((_BPW,), jnp.int32),
            pltpu.SemaphoreType.DMA,
        ],
        compiler_params=pltpu.CompilerParams(
            use_tc_tiling_on_sc=False, needs_layout_passes=False
        ),
    )(packed, ridp)

    return res.reshape(_BATCH, _NF, 32)
